# Initial kernel scaffold; baseline (speedup 1.0000x reference)
import jax
import jax.numpy as jnp
from jax.experimental import pallas as pl


def kernel(z, edge_index, neg_edge_index, W1, b1, W2, b2, W3, b3):
    raise NotImplementedError("write your pallas kernel here")


# speedup vs baseline: 2.3087x; 2.3087x over previous
import functools

import jax
import jax.numpy as jnp
from jax import lax
from jax.experimental import pallas as pl
from jax.experimental.pallas import tpu as pltpu
from jax.experimental.pallas import tpu_sc as plsc

N_NODES = 10000
D = 256
E_POS = 160000
E_NEG = 800000
E_TOT = E_POS + E_NEG

_NC = 2
_NS = 16
_NW = _NC * _NS
_PER_W = E_TOT // _NW
_CH = 120
_N_CHUNKS = _PER_W // _CH


def _tables_body(z_ref, w1t_ref, w1b_ref, b1_ref, p_ref, q_ref):
    zb = z_ref[...]
    p_ref[...] = (
        jnp.dot(zb, w1t_ref[...], preferred_element_type=jnp.float32) + b1_ref[...]
    )
    q_ref[...] = jnp.dot(zb, w1b_ref[...], preferred_element_type=jnp.float32)


def _make_tables(z, w1t, w1b, b1row):
    blk = 1000
    grid = N_NODES // blk
    return pl.pallas_call(
        _tables_body,
        grid=(grid,),
        in_specs=[
            pl.BlockSpec((blk, D), lambda i: (i, 0)),
            pl.BlockSpec((D, D), lambda i: (0, 0)),
            pl.BlockSpec((D, D), lambda i: (0, 0)),
            pl.BlockSpec((1, D), lambda i: (0, 0)),
        ],
        out_specs=[
            pl.BlockSpec((blk, D), lambda i: (i, 0)),
            pl.BlockSpec((blk, D), lambda i: (i, 0)),
        ],
        out_shape=[
            jax.ShapeDtypeStruct((N_NODES, D), jnp.float32),
            jax.ShapeDtypeStruct((N_NODES, D), jnp.float32),
        ],
    )(z, w1t, w1b, b1row)


def _gather_body(p_hbm, q_hbm, src_hbm, dst_hbm, out_hbm,
                 sidx, didx, prow, qrow, psem, qsem):
    wid = lax.axis_index("s") * _NC + lax.axis_index("c")
    base = wid * _PER_W

    def chunk(ci, carry):
        off = base + ci * _CH
        pltpu.sync_copy(src_hbm.at[pl.ds(off, _CH)], sidx)
        pltpu.sync_copy(dst_hbm.at[pl.ds(off, _CH)], didx)
        cp = pltpu.async_copy(p_hbm.at[sidx], prow, psem)
        cq = pltpu.async_copy(q_hbm.at[didx], qrow, qsem)
        cp.wait()
        cq.wait()

        def row_add(r, c2):
            for j in range(D // 16):
                s = pl.ds(j * 16, 16)
                prow[r, s] = prow[r, s] + qrow[r, s]
            return c2

        lax.fori_loop(0, _CH, row_add, 0)
        pltpu.sync_copy(prow, out_hbm.at[pl.ds(off, _CH)])
        return carry

    lax.fori_loop(0, _N_CHUNKS, chunk, 0)


def _gather_add(p, q, src, dst):
    mesh = plsc.VectorSubcoreMesh(core_axis_name="c", subcore_axis_name="s")
    fn = pl.kernel(
        _gather_body,
        out_type=jax.ShapeDtypeStruct((E_TOT, D), jnp.float32),
        mesh=mesh,
        scratch_types=[
            pltpu.VMEM((_CH,), jnp.int32),
            pltpu.VMEM((_CH,), jnp.int32),
            pltpu.VMEM((_CH, D), jnp.float32),
            pltpu.VMEM((_CH, D), jnp.float32),
            pltpu.SemaphoreType.DMA,
            pltpu.SemaphoreType.DMA,
        ],
    )
    return fn(p, q, src, dst)


_BE = 1600
_N_BLK = E_TOT // _BE
_POS_BLK = E_POS // _BE


def _loss_body(s_ref, w2_ref, b2_ref, w3_ref, b3_ref, out_ref):
    i = pl.program_id(0)
    x = jnp.maximum(s_ref[...], 0.0)
    h = jnp.maximum(
        jnp.dot(x, w2_ref[...], preferred_element_type=jnp.float32) + b2_ref[...],
        0.0,
    )
    logit = jnp.sum(h * w3_ref[...], axis=1, keepdims=True) + b3_ref[0, 0]
    sign = jnp.where(i < _POS_BLK, -1.0, 1.0)
    w = jnp.where(i < _POS_BLK, 0.5 / E_POS, 0.5 / E_NEG)
    y = sign * logit
    sp = jnp.maximum(y, 0.0) + jnp.log1p(jnp.exp(-jnp.abs(y)))
    blk = jnp.sum(sp) * w

    @pl.when(i == 0)
    def _():
        out_ref[0, 0] = 0.0

    out_ref[0, 0] += blk


def _loss(s, w2, b2row, w3row, b3m):
    return pl.pallas_call(
        _loss_body,
        grid=(_N_BLK,),
        in_specs=[
            pl.BlockSpec((_BE, D), lambda i: (i, 0)),
            pl.BlockSpec((D, D // 2), lambda i: (0, 0)),
            pl.BlockSpec((1, D // 2), lambda i: (0, 0)),
            pl.BlockSpec((1, D // 2), lambda i: (0, 0)),
            pl.BlockSpec(memory_space=pltpu.SMEM),
        ],
        out_specs=pl.BlockSpec(memory_space=pltpu.SMEM),
        out_shape=jax.ShapeDtypeStruct((1, 1), jnp.float32),
    )(s, w2, b2row, w3row, b3m)


def kernel(z, edge_index, neg_edge_index, W1, b1, W2, b2, W3, b3):
    src = jnp.concatenate([edge_index[0], neg_edge_index[0]]).astype(jnp.int32)
    dst = jnp.concatenate([edge_index[1], neg_edge_index[1]]).astype(jnp.int32)
    w1t = W1[:D]
    w1b = W1[D:]
    p, q = _make_tables(z, w1t, w1b, b1.reshape(1, D))
    s = _gather_add(p, q, src, dst)
    out = _loss(s, W2, b2.reshape(1, D // 2), W3.reshape(1, D // 2),
                b3.reshape(1, 1))
    return out[0, 0]
